# Initial kernel scaffold; baseline (speedup 1.0000x reference)
import functools

import jax
import jax.numpy as jnp
from jax import lax
from jax.experimental import pallas as pl
from jax.experimental.pallas import tpu as pltpu
from jax.experimental.pallas import tpu_sc as plsc

_BATCH = 16384
_HIST = 50
_DIM = 64
_NC = 2
_NS = 16
_NW = _NC * _NS
_B_TOTAL = _BATCH * _HIST
_B_PER_W = _B_TOTAL // _NW
_CHUNK = 128
_NCHUNKS = _B_PER_W // _CHUNK
_NBUF = 2


def _embed_body(weight_hbm, idx_hbm, out_hbm, idx_v, rows_v, sem0, sem1):
    wid = lax.axis_index("s") * _NC + lax.axis_index("c")
    pltpu.sync_copy(idx_hbm.at[wid], idx_v)
    sems = (sem0, sem1)

    def _start(g, b):
        pltpu.async_copy(weight_hbm.at[idx_v.at[g]], rows_v.at[b], sems[b])

    def _wait(g, b):
        pltpu.make_async_copy(
            weight_hbm.at[idx_v.at[g]], rows_v.at[b], sems[b]
        ).wait()

    _start(0, 0)

    def body(i, carry):
        for b in range(_NBUF):
            g = i * _NBUF + b
            nxt = g + 1

            @pl.when(nxt < _NCHUNKS)
            def _():
                _start(nxt, (b + 1) % _NBUF)

            _wait(g, b)
            pltpu.sync_copy(rows_v.at[b], out_hbm.at[wid, g])
        return carry

    lax.fori_loop(0, _NCHUNKS // _NBUF, body, 0)


@functools.partial(
    pl.kernel,
    out_type=jax.ShapeDtypeStruct((_NW, _NCHUNKS, _CHUNK, _DIM), jnp.float32),
    mesh=plsc.VectorSubcoreMesh(core_axis_name="c", subcore_axis_name="s"),
    scratch_types=[
        pltpu.VMEM((_NCHUNKS, _CHUNK), jnp.int32),
        pltpu.VMEM((_NBUF, _CHUNK, _DIM), jnp.float32),
        pltpu.SemaphoreType.DMA,
        pltpu.SemaphoreType.DMA,
    ],
)
def _embed_kernel(weight_hbm, idx_hbm, out_hbm, idx_v, rows_v, sem0, sem1):
    _embed_body(weight_hbm, idx_hbm, out_hbm, idx_v, rows_v, sem0, sem1)


def kernel(input_, weight):
    idx = input_.reshape(_NW, _NCHUNKS, _CHUNK)
    out = _embed_kernel(weight, idx)
    return out.reshape(_BATCH, _HIST, _DIM)


# speedup vs baseline: 1.8180x; 1.8180x over previous
import functools

import jax
import jax.numpy as jnp
from jax import lax
from jax.experimental import pallas as pl
from jax.experimental.pallas import tpu as pltpu
from jax.experimental.pallas import tpu_sc as plsc

_BATCH = 16384
_HIST = 50
_DIM = 64
_NC = 2
_NS = 16
_NW = _NC * _NS
_B_TOTAL = _BATCH * _HIST
_B_PER_W = _B_TOTAL // _NW
_CHUNK = 128
_NCHUNKS = _B_PER_W // _CHUNK
_NBUF = 2


def _embed_body(weight_hbm, idx_hbm, out_hbm, idx_v, rows_v, sem0, sem1):
    wid = lax.axis_index("s") * _NC + lax.axis_index("c")
    pltpu.sync_copy(idx_hbm.at[wid], idx_v)
    sems = (sem0, sem1)

    def _start(g, b):
        pltpu.async_copy(weight_hbm.at[idx_v.at[g]], rows_v.at[b], sems[b])

    def _wait(g, b):
        pltpu.make_async_copy(
            weight_hbm.at[idx_v.at[g]], rows_v.at[b], sems[b]
        ).wait()

    _start(0, 0)

    def body(i, carry):
        for b in range(_NBUF):
            g = i * _NBUF + b
            nxt = g + 1

            @pl.when(nxt < _NCHUNKS)
            def _():
                _start(nxt, (b + 1) % _NBUF)

            _wait(g, b)
            pltpu.sync_copy(rows_v.at[b], out_hbm.at[wid, g])
        return carry

    lax.fori_loop(0, _NCHUNKS // _NBUF, body, 0)


@functools.partial(
    pl.kernel,
    out_type=jax.ShapeDtypeStruct((_NW, _NCHUNKS, _CHUNK, _DIM), jnp.float32),
    mesh=plsc.VectorSubcoreMesh(core_axis_name="c", subcore_axis_name="s"),
    compiler_params=pltpu.CompilerParams(use_tc_tiling_on_sc=False),
    scratch_types=[
        pltpu.VMEM((_NCHUNKS, _CHUNK), jnp.int32),
        pltpu.VMEM((_NBUF, _CHUNK, _DIM), jnp.float32),
        pltpu.SemaphoreType.DMA,
        pltpu.SemaphoreType.DMA,
    ],
)
def _embed_kernel(weight_hbm, idx_hbm, out_hbm, idx_v, rows_v, sem0, sem1):
    _embed_body(weight_hbm, idx_hbm, out_hbm, idx_v, rows_v, sem0, sem1)


def kernel(input_, weight):
    idx = input_.reshape(_NW, _NCHUNKS, _CHUNK)
    out = _embed_kernel(weight, idx)
    return out.reshape(_BATCH, _HIST, _DIM)


# speedup vs baseline: 1.8899x; 1.0396x over previous
import functools

import jax
import jax.numpy as jnp
from jax import lax
from jax.experimental import pallas as pl
from jax.experimental.pallas import tpu as pltpu
from jax.experimental.pallas import tpu_sc as plsc

_BATCH = 16384
_HIST = 50
_DIM = 64
_NC = 2
_NS = 16
_NW = _NC * _NS
_B_TOTAL = _BATCH * _HIST
_B_PER_W = _B_TOTAL // _NW
_CHUNK = 128
_SUB = 4
_ROWS = _CHUNK * _SUB
_M = _B_PER_W // _ROWS
_NBUF = 2


def _embed_body(weight_hbm, idx_hbm, out_hbm, idx_v, rows_v,
                gsem0, gsem1, ssem0, ssem1):
    wid = lax.axis_index("s") * _NC + lax.axis_index("c")
    pltpu.sync_copy(idx_hbm.at[wid], idx_v)
    gsems = (gsem0, gsem1)
    ssems = (ssem0, ssem1)

    def _start_fill(g, b):
        for j in range(_SUB):
            pltpu.async_copy(
                weight_hbm.at[idx_v.at[g, j]],
                rows_v.at[b, pl.ds(j * _CHUNK, _CHUNK)],
                gsems[b],
            )

    def _drain_fill(b):
        pltpu.make_async_copy(
            weight_hbm.at[pl.ds(0, _ROWS)], rows_v.at[b], gsems[b]
        ).wait()

    def _start_store(g, b):
        pltpu.async_copy(rows_v.at[b], out_hbm.at[wid, g], ssems[b])

    def _drain_store(g, b):
        pltpu.make_async_copy(
            rows_v.at[b], out_hbm.at[wid, g], ssems[b]
        ).wait()

    _start_fill(0, 0)

    def body(i, carry):
        for b in range(_NBUF):
            g = i * _NBUF + b
            nxt = g + 1
            nb = (b + 1) % _NBUF

            @pl.when(nxt < _M)
            def _():
                @pl.when(nxt >= _NBUF)
                def _():
                    _drain_store(nxt - _NBUF, nb)

                _start_fill(nxt, nb)

            _drain_fill(b)
            _start_store(g, b)
        return carry

    lax.fori_loop(0, _M // _NBUF, body, 0)
    _drain_store(_M - 2, (_M - 2) % _NBUF)
    _drain_store(_M - 1, (_M - 1) % _NBUF)


@functools.partial(
    pl.kernel,
    out_type=jax.ShapeDtypeStruct((_NW, _M, _ROWS, _DIM), jnp.float32),
    mesh=plsc.VectorSubcoreMesh(core_axis_name="c", subcore_axis_name="s"),
    compiler_params=pltpu.CompilerParams(use_tc_tiling_on_sc=False),
    scratch_types=[
        pltpu.VMEM((_M, _SUB, _CHUNK), jnp.int32),
        pltpu.VMEM((_NBUF, _ROWS, _DIM), jnp.float32),
        pltpu.SemaphoreType.DMA,
        pltpu.SemaphoreType.DMA,
        pltpu.SemaphoreType.DMA,
        pltpu.SemaphoreType.DMA,
    ],
)
def _embed_kernel(weight_hbm, idx_hbm, out_hbm, idx_v, rows_v,
                  gsem0, gsem1, ssem0, ssem1):
    _embed_body(weight_hbm, idx_hbm, out_hbm, idx_v, rows_v,
                gsem0, gsem1, ssem0, ssem1)


def kernel(input_, weight):
    idx = input_.reshape(_NW, _M, _SUB, _CHUNK)
    out = _embed_kernel(weight, idx)
    return out.reshape(_BATCH, _HIST, _DIM)
